# Initial kernel scaffold; baseline (speedup 1.0000x reference)
import jax
import jax.numpy as jnp
from jax.experimental import pallas as pl


def kernel(A, X, W1, W_mlp, b_mlp):
    raise NotImplementedError("write your pallas kernel here")


# speedup vs baseline: 13.6661x; 13.6661x over previous
import jax
import jax.numpy as jnp
from jax import lax
from jax.experimental import pallas as pl
from jax.experimental.pallas import tpu as pltpu
from jax.experimental.pallas import tpu_sc as plsc

N = 512
DH = 128
E = N * N
K_KEEP = max(1, int(0.1 * E))
INT_MIN = -(2 ** 31)
ROWS_PER_STEP = 8

SC_CORES = 2
SC_SUBCORES = 16
SC_WORKERS = SC_CORES * SC_SUBCORES
CHUNK = E // SC_WORKERS
IDX_BLK = 128
BLKS = CHUNK // IDX_BLK
FIRE = 16


def _round_bf16(x):
    bits = lax.bitcast_convert_type(x, jnp.int32)
    r = bits + jnp.int32(0x7FFF) + lax.shift_right_logical(bits, 16) % 2
    return lax.bitcast_convert_type(r & jnp.int32(-65536), jnp.float32)


def _dense_stage_body(a_ref, x_ref, w1_ref, whr_ref, wg_ref, wlb_ref,
                      h_ref, rank_ref, c_ref, y00_ref, z_ref):
    f32 = jnp.float32
    bf16 = jnp.bfloat16

    bb = jnp.dot(x_ref[...].astype(bf16), w1_ref[...].astype(bf16),
                 preferred_element_type=f32).astype(bf16)
    a_bf = a_ref[...].astype(bf16)
    hvt = jnp.maximum(
        lax.dot_general(bb, a_bf, (((0,), (1,)), ((), ())),
                        preferred_element_type=f32), 0.0)
    ht = _round_bf16(hvt)
    h_ref[...] = ht

    whr = whr_ref[...]
    colsum = jnp.dot(hvt, jnp.full((N, 1), 1.0, f32),
                     preferred_element_type=f32)
    hgb = _round_bf16(colsum * (1.0 / N))
    c = jnp.sum(hgb * wg_ref[...].reshape(DH, 1)) + wlb_ref[0, 0]
    c_ref[0, 0] = c
    s0 = jnp.sum(ht[:, 0:1] * whr.reshape(DH, 1))
    y00_ref[0, 0] = s0 + s0 + c

    mf = (a_ref[...] != 0.0).astype(f32)
    ir = lax.broadcasted_iota(jnp.int32, (N, N), 0)
    ic = lax.broadcasted_iota(jnp.int32, (N, N), 1)
    ut = (ir <= ic).astype(f32)
    sl = (ic < ir).astype(f32)
    rowcum = jnp.dot(mf, ut, preferred_element_type=f32)
    colpre = jnp.dot(sl, mf, preferred_element_type=f32)
    rowoff = jnp.sum(colpre, axis=1, keepdims=True)
    rank_ref[...] = (rowoff + rowcum - mf).astype(jnp.int32)
    z_ref[0, 0] = jnp.sum(mf).astype(jnp.int32)


def _score_rows_body(htfull_ref, whr_ref, y_ref):
    f32 = jnp.float32
    htfull = htfull_ref[...]
    whr = whr_ref[...]
    j0 = pl.program_id(0) * ROWS_PER_STEP
    ir = lax.broadcasted_iota(jnp.int32, (N, ROWS_PER_STEP), 0)
    ic = lax.broadcasted_iota(jnp.int32, (N, ROWS_PER_STEP), 1)
    onehot = (ir == j0 + ic).astype(f32)
    hcols = jnp.dot(htfull, onehot, preferred_element_type=f32)
    for jl in range(ROWS_PER_STEP):
        hsum_t = _round_bf16(hcols[:, jl:jl + 1] + htfull)
        y_ref[jl:jl + 1, :] = jnp.dot(whr, hsum_t,
                                      preferred_element_type=f32,
                                      precision=lax.Precision.HIGHEST)


def _select_stage_body(a_ref, y_ref, gal_ref, g2_ref,
                       c_ref, y00_ref, z_ref, out_ref):
    f32 = jnp.float32
    a = a_ref[...]
    mask = a != 0.0
    c = c_ref[0, 0]
    score = (y_ref[...] + c) + gal_ref[...]

    def orderable(x):
        bits = lax.bitcast_convert_type(x, jnp.int32)
        return jnp.where(bits >= 0, bits, bits ^ jnp.int32(0x7FFFFFFF))

    imin = jnp.int32(INT_MIN)
    su = jnp.where(mask, orderable(score), imin)

    ir = lax.broadcasted_iota(jnp.int32, (N, N), 0)
    ic = lax.broadcasted_iota(jnp.int32, (N, N), 1)
    eidx = ir * N + ic
    validph = eidx >= z_ref[0, 0]
    sph = jnp.where(validph, orderable(y00_ref[0, 0] + g2_ref[...]), imin)

    kk = jnp.int32(K_KEEP)

    def search_body(t, thr):
        bit = jnp.int32(31) - t
        cand = thr + lax.shift_left(jnp.int32(1), bit)
        cnt = (jnp.sum((su >= cand).astype(jnp.int32))
               + jnp.sum((sph >= cand).astype(jnp.int32)))
        return jnp.where(cnt >= kk, cand, thr)

    thr = lax.fori_loop(0, 32, search_body, imin)

    ph_gt = jnp.sum((sph > thr).astype(jnp.int32))
    c_gt = jnp.sum((su > thr).astype(jnp.int32)) + ph_gt
    need = (kk - c_gt).astype(f32)

    tie = mask & (su == thr)
    tf = tie.astype(f32)
    ut = (ir <= ic).astype(f32)
    sl = (ic < ir).astype(f32)
    trowcum = jnp.dot(tf, ut, preferred_element_type=f32)
    tcolpre = jnp.dot(sl, tf, preferred_element_type=f32)
    trank = jnp.sum(tcolpre, axis=1, keepdims=True) + trowcum - tf
    n_tie_real = jnp.sum(tf)
    sel_tie = tie & (trank < need)

    phantom_sel = (ph_gt > 0) | (need > n_tie_real)
    sel = mask & ((su > thr) | sel_tie)
    out = jnp.where(sel, a, 0.0)
    out_ref[...] = jnp.where((eidx == 0) & phantom_sel, a, out)


def _dense_stage(A, X, W1, whr, wg, wlb):
    f32 = jnp.float32
    smem11 = pl.BlockSpec(memory_space=pltpu.SMEM)
    return pl.pallas_call(
        _dense_stage_body,
        out_shape=[
            jax.ShapeDtypeStruct((DH, N), f32),
            jax.ShapeDtypeStruct((N, N), jnp.int32),
            jax.ShapeDtypeStruct((1, 1), f32),
            jax.ShapeDtypeStruct((1, 1), f32),
            jax.ShapeDtypeStruct((1, 1), jnp.int32),
        ],
        in_specs=[pl.BlockSpec(), pl.BlockSpec(), pl.BlockSpec(),
                  pl.BlockSpec(), pl.BlockSpec(), smem11],
        out_specs=[pl.BlockSpec(), pl.BlockSpec(),
                   smem11, smem11, smem11],
    )(A, X, W1, whr, wg, wlb)


def _score_rows(ht, whr):
    f32 = jnp.float32
    return pl.pallas_call(
        _score_rows_body,
        grid=(N // ROWS_PER_STEP,),
        out_shape=jax.ShapeDtypeStruct((N, N), f32),
        in_specs=[
            pl.BlockSpec((DH, N), lambda j: (0, 0)),
            pl.BlockSpec((1, DH), lambda j: (0, 0)),
        ],
        out_specs=pl.BlockSpec((ROWS_PER_STEP, N), lambda j: (j, 0)),
    )(ht, whr)


def _select_stage(A, y, gal, g2, c, y00, z):
    smem11 = pl.BlockSpec(memory_space=pltpu.SMEM)
    return pl.pallas_call(
        _select_stage_body,
        out_shape=jax.ShapeDtypeStruct((N, N), jnp.float32),
        in_specs=[pl.BlockSpec(), pl.BlockSpec(), pl.BlockSpec(),
                  pl.BlockSpec(), smem11, smem11, smem11],
        out_specs=pl.BlockSpec(),
    )(A, y, gal, g2, c, y00, z)


def _sc_gather_body(rank_hbm, g_hbm, out_hbm, idx_v, val_v, sem):
    wid = lax.axis_index("s") * SC_CORES + lax.axis_index("c")
    base = wid * CHUNK
    pltpu.sync_copy(rank_hbm.at[pl.ds(base, CHUNK)], idx_v)

    def outer(o, carry):
        copies = []
        for f in range(FIRE):
            j = o * FIRE + f
            cp = pltpu.make_async_copy(
                g_hbm.at[idx_v.at[pl.ds(j * IDX_BLK, IDX_BLK)]],
                val_v.at[pl.ds(j * IDX_BLK, IDX_BLK)],
                sem)
            cp.start()
            copies.append(cp)
        for cp in copies:
            cp.wait()
        return carry

    lax.fori_loop(0, BLKS // FIRE, outer, 0)
    pltpu.sync_copy(val_v, out_hbm.at[pl.ds(base, CHUNK)])


def _sc_gather(rank_flat, gflat):
    call = pl.kernel(
        _sc_gather_body,
        mesh=plsc.VectorSubcoreMesh(core_axis_name="c", subcore_axis_name="s"),
        out_type=jax.ShapeDtypeStruct((E,), jnp.float32),
        scratch_types=[
            pltpu.VMEM((CHUNK,), jnp.int32),
            pltpu.VMEM((CHUNK,), jnp.float32),
            pltpu.SemaphoreType.DMA,
        ],
    )
    return call(rank_flat, gflat)


def kernel(A, X, W1, W_mlp, b_mlp):
    f32 = jnp.float32
    gkey = jax.random.fold_in(jax.random.key(0), 7)
    u = jax.random.uniform(gkey, (E, 1), dtype=f32)
    g = -jnp.log(-jnp.log(u + 1e-08) + 1e-08)
    gflat = g[:, 0]

    whr = W_mlp[:DH].reshape(1, DH)
    wg = W_mlp[DH:2 * DH].reshape(1, DH)
    wlb = (W_mlp[2 * DH, 0] + b_mlp[0]).reshape(1, 1)

    gflat, A_b, X_b, W1_b = lax.optimization_barrier((gflat, A, X, W1))
    ht, rank, c, y00, z = _dense_stage(A_b, X_b, W1_b, whr, wg, wlb)
    y = _score_rows(ht, whr)
    y, rank = lax.optimization_barrier((y, rank))
    gal = _sc_gather(rank.reshape(E), gflat)
    y, gal, A_s = lax.optimization_barrier((y, gal, A))
    return _select_stage(A_s, y, gal.reshape(N, N),
                         gflat.reshape(N, N), c, y00, z)
